# baseline (speedup 1.0000x reference)
import functools

import jax
import jax.numpy as jnp
import numpy as np
from jax import lax
from jax.experimental import pallas as pl
from jax.experimental.pallas import tpu as pltpu
from jax.experimental.pallas import tpu_sc as plsc

N = 10000
E = 320000
DIM = 128
H = 8
HD = 16

NC = 2
NS = 16
NW = NC * NS
EPT = E // NW
BLK = 40
CB = 5
NCH = EPT // (CB * BLK)
NPAIR = NCH // 2
RPT = N // NS

_ROW_BLK = 1000

_K16 = np.zeros((16, DIM), dtype=np.float32)
for _h in range(H):
    _K16[_h, _h * HD:(_h + 1) * HD] = 1.0

_PCOL = np.zeros((DIM,), dtype=np.int32)
for _t in range(H // 2):
    for _i in range(16):
        _PCOL[32 * _t + 2 * _i] = 32 * _t + _i
        _PCOL[32 * _t + 2 * _i + 1] = 32 * _t + 16 + _i


def _qkv_body(x_ref, w_ref, b_ref, q_ref, k_ref, v_ref):
    y = jnp.dot(x_ref[...], w_ref[...], preferred_element_type=jnp.float32)
    y = y + b_ref[...]
    q_ref[...] = y[:, 0:DIM].astype(jnp.bfloat16)
    k_ref[...] = y[:, DIM:2 * DIM].astype(jnp.bfloat16)
    v_ref[...] = y[:, 2 * DIM:3 * DIM]


_qkv_call = pl.pallas_call(
    _qkv_body,
    grid=(N // _ROW_BLK,),
    in_specs=[
        pl.BlockSpec((_ROW_BLK, DIM), lambda i: (i, 0)),
        pl.BlockSpec((DIM, 3 * DIM), lambda i: (0, 0)),
        pl.BlockSpec((1, 3 * DIM), lambda i: (0, 0)),
    ],
    out_specs=[pl.BlockSpec((_ROW_BLK, DIM), lambda i: (i, 0))] * 3,
    out_shape=[jax.ShapeDtypeStruct((N, DIM), jnp.bfloat16),
               jax.ShapeDtypeStruct((N, DIM), jnp.bfloat16),
               jax.ShapeDtypeStruct((N, DIM), jnp.float32)],
)


def _out_body(n0_ref, n1_ref, d0_ref, d1_ref, k16_ref, wo_ref, bo_ref, o_ref):
    den = d0_ref[...] + d1_ref[...]
    den_exp = jnp.dot(den, k16_ref[...], preferred_element_type=jnp.float32)
    agg = (n0_ref[...] + n1_ref[...]) / (den_exp + 1e-16)
    o_ref[...] = jnp.dot(agg, wo_ref[...],
                         preferred_element_type=jnp.float32) + bo_ref[...]


_out_call = pl.pallas_call(
    _out_body,
    grid=(N // _ROW_BLK,),
    in_specs=[
        pl.BlockSpec((_ROW_BLK, DIM), lambda i: (i, 0)),
        pl.BlockSpec((_ROW_BLK, DIM), lambda i: (i, 0)),
        pl.BlockSpec((_ROW_BLK, 16), lambda i: (i, 0)),
        pl.BlockSpec((_ROW_BLK, 16), lambda i: (i, 0)),
        pl.BlockSpec((16, DIM), lambda i: (0, 0)),
        pl.BlockSpec((DIM, DIM), lambda i: (0, 0)),
        pl.BlockSpec((1, DIM), lambda i: (0, 0)),
    ],
    out_specs=pl.BlockSpec((_ROW_BLK, DIM), lambda i: (i, 0)),
    out_shape=jax.ShapeDtypeStruct((N, DIM), jnp.float32),
)


def _edge_body(q_hbm, k_hbm, v_hbm, ei_hbm, num_out, den_out,
               isb0, isb1, idb0, idb1,
               qb0, qb1, kb0, kb1, vb0, vb1, wvb, wb,
               acc, dacc,
               sq0, sq1, sk0, sk1, sv0, sv1, ssn, ssd):
    c = lax.axis_index("c")
    s = lax.axis_index("s")
    wid = c * NS + s
    isb = (isb0, isb1)
    idb = (idb0, idb1)
    qbufs = (qb0, qb1)
    kbufs = (kb0, kb1)
    vbufs = (vb0, vb1)
    sq = (sq0, sq1)
    sk = (sk0, sk1)
    svm = (sv0, sv1)

    zero16 = jnp.zeros((16,), jnp.float32)

    def zrow(i, _):
        for j in range(DIM // 16):
            wvb[i, pl.ds(j * 16, 16)] = zero16
        wb[i, :] = zero16
        return 0

    lax.fori_loop(0, BLK, zrow, 0)
    r0 = s * RPT
    for j in range(RPT // BLK):
        pltpu.sync_copy(wvb, acc.at[pl.ds(r0 + j * BLK, BLK)])
        pltpu.sync_copy(wb, dacc.at[pl.ds(r0 + j * BLK, BLK)])
    rem = RPT - (RPT // BLK) * BLK
    pltpu.sync_copy(wvb.at[pl.ds(0, rem)],
                    acc.at[pl.ds(r0 + RPT - rem, rem)])
    pltpu.sync_copy(wb.at[pl.ds(0, rem)],
                    dacc.at[pl.ds(r0 + RPT - rem, rem)])
    plsc.subcore_barrier()

    idx16 = lax.iota(jnp.int32, 16)
    lane15 = jnp.where(idx16 < H, idx16, H - 1) * HD + (HD - 1)
    lane_h = [jnp.broadcast_to(jnp.int32(h), (16,)) for h in range(H)]

    def fire_gathers(sl, pos, gpar):
        return (
            pltpu.async_copy(q_hbm.at[isb[sl].at[pos]], qbufs[gpar],
                             sq[gpar]),
            pltpu.async_copy(k_hbm.at[idb[sl].at[pos]], kbufs[gpar],
                             sk[gpar]),
            pltpu.async_copy(v_hbm.at[isb[sl].at[pos]], vbufs[gpar],
                             svm[gpar]),
        )

    def compute(gpar):
        qbuf = qbufs[gpar]
        kbuf = kbufs[gpar]
        vbuf = vbufs[gpar]

        @plsc.parallel_loop(0, BLK, unroll=1)
        def _edge(e):
            wrow = zero16
            for t in range(H // 2):
                qp = qbuf[e, pl.ds(32 * t, 32)]
                kp = kbuf[e, pl.ds(32 * t, 32)]
                qa, qb2 = plsc.unpack(qp, format=plsc.PackFormat.INTERLEAVED)
                ka, kb2 = plsc.unpack(kp, format=plsc.PackFormat.INTERLEAVED)
                prod = qa * ka + qb2 * kb2
                s0 = jnp.sum(jnp.where(idx16 < 8, prod, 0.0))
                s1 = jnp.sum(jnp.where(idx16 < 8, 0.0, prod))
                w0 = jnp.exp(jnp.broadcast_to(s0 * 0.25, (HD,)))
                w1 = jnp.exp(jnp.broadcast_to(s1 * 0.25, (HD,)))
                wvb[e, pl.ds(32 * t, HD)] = (
                    vbuf[e, pl.ds(32 * t, HD)] * w0)
                wvb[e, pl.ds(32 * t + HD, HD)] = (
                    vbuf[e, pl.ds(32 * t + HD, HD)] * w1)
                wrow = jnp.where(idx16 == 2 * t, w0, wrow)
                wrow = jnp.where(idx16 == 2 * t + 1, w1, wrow)
            wb[e, :] = wrow

    def fire_scatter(sl, pos):
        return (
            pltpu.async_copy(wvb, acc.at[idb[sl].at[pos]], ssn, add=True),
            pltpu.async_copy(wb, dacc.at[idb[sl].at[pos]], ssd, add=True),
        )

    schedule = [(0, p) for p in range(CB)] + [(1, p) for p in range(CB)]

    def pair_body(p, _):
        gch = wid * NCH + 2 * p
        pltpu.sync_copy(ei_hbm.at[0, gch], idb[0])
        pltpu.sync_copy(ei_hbm.at[1, gch], isb[0])
        pltpu.sync_copy(ei_hbm.at[0, gch + 1], idb[1])
        pltpu.sync_copy(ei_hbm.at[1, gch + 1], isb[1])
        gath = fire_gathers(0, 0, 0)
        scat = None
        for j, (sl, pos) in enumerate(schedule):
            gpar = j % 2
            nxt = None
            if j + 1 < len(schedule):
                nsl, npos = schedule[j + 1]
                nxt = fire_gathers(nsl, npos, 1 - gpar)
            for d in gath:
                d.wait()
            if scat is not None:
                for d in scat:
                    d.wait()
            compute(gpar)
            scat = fire_scatter(sl, pos)
            gath = nxt
        for d in scat:
            d.wait()
        return 0

    lax.fori_loop(0, NPAIR, pair_body, 0)
    plsc.subcore_barrier()

    @pl.when(s == 0)
    def _writeback():
        pltpu.sync_copy(acc, num_out.at[c])
        pltpu.sync_copy(dacc, den_out.at[c])


_edge_call = pl.kernel(
    _edge_body,
    out_type=(
        jax.ShapeDtypeStruct((NC, N, DIM), jnp.float32),
        jax.ShapeDtypeStruct((NC, N, 16), jnp.float32),
    ),
    mesh=plsc.VectorSubcoreMesh(core_axis_name="c", subcore_axis_name="s"),
    compiler_params=pltpu.CompilerParams(needs_layout_passes=False,
                                         use_tc_tiling_on_sc=False),
    scratch_types=[
        pltpu.VMEM((CB, BLK), jnp.int32),
        pltpu.VMEM((CB, BLK), jnp.int32),
        pltpu.VMEM((CB, BLK), jnp.int32),
        pltpu.VMEM((CB, BLK), jnp.int32),
        pltpu.VMEM((BLK, DIM), jnp.bfloat16),
        pltpu.VMEM((BLK, DIM), jnp.bfloat16),
        pltpu.VMEM((BLK, DIM), jnp.bfloat16),
        pltpu.VMEM((BLK, DIM), jnp.bfloat16),
        pltpu.VMEM((BLK, DIM), jnp.float32),
        pltpu.VMEM((BLK, DIM), jnp.float32),
        pltpu.VMEM((BLK, DIM), jnp.float32),
        pltpu.VMEM((BLK, 16), jnp.float32),
        pltpu.VMEM_SHARED((N, DIM), jnp.float32),
        pltpu.VMEM_SHARED((N, 16), jnp.float32),
        pltpu.SemaphoreType.DMA,
        pltpu.SemaphoreType.DMA,
        pltpu.SemaphoreType.DMA,
        pltpu.SemaphoreType.DMA,
        pltpu.SemaphoreType.DMA,
        pltpu.SemaphoreType.DMA,
        pltpu.SemaphoreType.DMA,
        pltpu.SemaphoreType.DMA,
    ],
)


def kernel(x, edge_index, Wq, bq, Wk, bk, Wv, bv, Wo, bo):
    w_all = jnp.concatenate([Wq, Wk, Wv], axis=1)
    b_all = jnp.concatenate([bq, bk, bv]).reshape(1, 3 * DIM)
    q, k, v = _qkv_call(x, w_all, b_all)
    ei4 = edge_index.reshape(2, NW * NCH, CB, BLK)
    num, den = _edge_call(q, k, v, ei4)
    out = _out_call(num[0], num[1], den[0], den[1],
                    jnp.asarray(_K16), Wo, bo.reshape(1, DIM))
    return out


# speedup vs baseline: 1.2985x; 1.2985x over previous
import functools

import jax
import jax.numpy as jnp
import numpy as np
from jax import lax
from jax.experimental import pallas as pl
from jax.experimental.pallas import tpu as pltpu
from jax.experimental.pallas import tpu_sc as plsc

N = 10000
E = 320000
DIM = 128
H = 8
HD = 16

NC = 2
NS = 16
NW = NC * NS
EPT = E // NW
BLK = 40
CB = 5
NCH = EPT // (CB * BLK)
NPAIR = NCH // 2
RPT = N // NS

_ROW_BLK = 1000

_K16 = np.zeros((16, DIM), dtype=np.float32)
for _h in range(H):
    _K16[_h, _h * HD:(_h + 1) * HD] = 1.0

_PCOL = np.zeros((DIM,), dtype=np.int32)
for _t in range(H // 2):
    for _i in range(16):
        _PCOL[32 * _t + 2 * _i] = 32 * _t + _i
        _PCOL[32 * _t + 2 * _i + 1] = 32 * _t + 16 + _i


def _qkv_body(x_ref, w_ref, b_ref, q_ref, k_ref, v_ref):
    y = jnp.dot(x_ref[...], w_ref[...], preferred_element_type=jnp.float32)
    y = y + b_ref[...]
    q_ref[...] = y[:, 0:DIM]
    k_ref[...] = y[:, DIM:2 * DIM]
    v_ref[...] = y[:, 2 * DIM:3 * DIM]


_qkv_call = pl.pallas_call(
    _qkv_body,
    grid=(N // _ROW_BLK,),
    in_specs=[
        pl.BlockSpec((_ROW_BLK, DIM), lambda i: (i, 0)),
        pl.BlockSpec((DIM, 3 * DIM), lambda i: (0, 0)),
        pl.BlockSpec((1, 3 * DIM), lambda i: (0, 0)),
    ],
    out_specs=[pl.BlockSpec((_ROW_BLK, DIM), lambda i: (i, 0))] * 3,
    out_shape=[jax.ShapeDtypeStruct((N, DIM), jnp.float32)] * 3,
)


def _out_body(n0_ref, n1_ref, d0_ref, d1_ref, k16_ref, wo_ref, bo_ref, o_ref):
    den = d0_ref[...] + d1_ref[...]
    den_exp = jnp.dot(den, k16_ref[...], preferred_element_type=jnp.float32)
    agg = (n0_ref[...] + n1_ref[...]) / (den_exp + 1e-16)
    o_ref[...] = jnp.dot(agg, wo_ref[...],
                         preferred_element_type=jnp.float32) + bo_ref[...]


_out_call = pl.pallas_call(
    _out_body,
    grid=(N // _ROW_BLK,),
    in_specs=[
        pl.BlockSpec((_ROW_BLK, DIM), lambda i: (i, 0)),
        pl.BlockSpec((_ROW_BLK, DIM), lambda i: (i, 0)),
        pl.BlockSpec((_ROW_BLK, 16), lambda i: (i, 0)),
        pl.BlockSpec((_ROW_BLK, 16), lambda i: (i, 0)),
        pl.BlockSpec((16, DIM), lambda i: (0, 0)),
        pl.BlockSpec((DIM, DIM), lambda i: (0, 0)),
        pl.BlockSpec((1, DIM), lambda i: (0, 0)),
    ],
    out_specs=pl.BlockSpec((_ROW_BLK, DIM), lambda i: (i, 0)),
    out_shape=jax.ShapeDtypeStruct((N, DIM), jnp.float32),
)


def _edge_body(q_hbm, k_hbm, v_hbm, ei_hbm, num_out, den_out,
               isb0, isb1, idb0, idb1,
               qb0, qb1, kb0, kb1, vb0, vb1, wvb, wb,
               acc, dacc,
               sq0, sq1, sk0, sk1, sv0, sv1, ssn, ssd):
    c = lax.axis_index("c")
    s = lax.axis_index("s")
    wid = c * NS + s
    isb = (isb0, isb1)
    idb = (idb0, idb1)
    qbufs = (qb0, qb1)
    kbufs = (kb0, kb1)
    vbufs = (vb0, vb1)
    sq = (sq0, sq1)
    sk = (sk0, sk1)
    svm = (sv0, sv1)

    zero16 = jnp.zeros((16,), jnp.float32)

    def zrow(i, _):
        for j in range(DIM // 16):
            wvb[i, pl.ds(j * 16, 16)] = zero16
        wb[i, :] = zero16
        return 0

    lax.fori_loop(0, BLK, zrow, 0)
    r0 = s * RPT
    for j in range(RPT // BLK):
        pltpu.sync_copy(wvb, acc.at[pl.ds(r0 + j * BLK, BLK)])
        pltpu.sync_copy(wb, dacc.at[pl.ds(r0 + j * BLK, BLK)])
    rem = RPT - (RPT // BLK) * BLK
    pltpu.sync_copy(wvb.at[pl.ds(0, rem)],
                    acc.at[pl.ds(r0 + RPT - rem, rem)])
    pltpu.sync_copy(wb.at[pl.ds(0, rem)],
                    dacc.at[pl.ds(r0 + RPT - rem, rem)])
    plsc.subcore_barrier()

    idx16 = lax.iota(jnp.int32, 16)
    lane15 = jnp.where(idx16 < H, idx16, H - 1) * HD + (HD - 1)
    lane_h = [jnp.broadcast_to(jnp.int32(h), (16,)) for h in range(H)]

    def fire_gathers(sl, pos, gpar):
        return (
            pltpu.async_copy(q_hbm.at[isb[sl].at[pos]], qbufs[gpar],
                             sq[gpar]),
            pltpu.async_copy(k_hbm.at[idb[sl].at[pos]], kbufs[gpar],
                             sk[gpar]),
            pltpu.async_copy(v_hbm.at[isb[sl].at[pos]], vbufs[gpar],
                             svm[gpar]),
        )

    def compute(gpar):
        qbuf = qbufs[gpar]
        kbuf = kbufs[gpar]
        vbuf = vbufs[gpar]

        @plsc.parallel_loop(0, BLK, unroll=2)
        def _edge(e):
            wrow = zero16
            wex = []
            for h in range(H):
                p = qbuf[e, pl.ds(h * HD, HD)] * kbuf[e, pl.ds(h * HD, HD)]
                bc = jnp.broadcast_to(jnp.sum(p) * 0.25, (HD,))
                we = jnp.exp(bc)
                wex.append(we)
                wrow = jnp.where(idx16 == h, we, wrow)
            wb[e, :] = wrow
            for h in range(H):
                wvb[e, pl.ds(h * HD, HD)] = (
                    vbuf[e, pl.ds(h * HD, HD)] * wex[h])

    def fire_scatter(sl, pos):
        return (
            pltpu.async_copy(wvb, acc.at[idb[sl].at[pos]], ssn, add=True),
            pltpu.async_copy(wb, dacc.at[idb[sl].at[pos]], ssd, add=True),
        )

    schedule = [(0, p) for p in range(CB)] + [(1, p) for p in range(CB)]

    def pair_body(p, _):
        gch = wid * NCH + 2 * p
        pltpu.sync_copy(ei_hbm.at[0, gch], idb[0])
        pltpu.sync_copy(ei_hbm.at[1, gch], isb[0])
        pltpu.sync_copy(ei_hbm.at[0, gch + 1], idb[1])
        pltpu.sync_copy(ei_hbm.at[1, gch + 1], isb[1])
        gath = fire_gathers(0, 0, 0)
        scat = None
        for j, (sl, pos) in enumerate(schedule):
            gpar = j % 2
            nxt = None
            if j + 1 < len(schedule):
                nsl, npos = schedule[j + 1]
                nxt = fire_gathers(nsl, npos, 1 - gpar)
            for d in gath:
                d.wait()
            if scat is not None:
                for d in scat:
                    d.wait()
            compute(gpar)
            scat = fire_scatter(sl, pos)
            gath = nxt
        for d in scat:
            d.wait()
        return 0

    lax.fori_loop(0, NPAIR, pair_body, 0)
    plsc.subcore_barrier()

    @pl.when(s == 0)
    def _writeback():
        pltpu.sync_copy(acc, num_out.at[c])
        pltpu.sync_copy(dacc, den_out.at[c])


_edge_call = pl.kernel(
    _edge_body,
    out_type=(
        jax.ShapeDtypeStruct((NC, N, DIM), jnp.float32),
        jax.ShapeDtypeStruct((NC, N, 16), jnp.float32),
    ),
    mesh=plsc.VectorSubcoreMesh(core_axis_name="c", subcore_axis_name="s"),
    compiler_params=pltpu.CompilerParams(needs_layout_passes=False,
                                         use_tc_tiling_on_sc=False),
    scratch_types=[
        pltpu.VMEM((CB, BLK), jnp.int32),
        pltpu.VMEM((CB, BLK), jnp.int32),
        pltpu.VMEM((CB, BLK), jnp.int32),
        pltpu.VMEM((CB, BLK), jnp.int32),
        pltpu.VMEM((BLK, DIM), jnp.float32),
        pltpu.VMEM((BLK, DIM), jnp.float32),
        pltpu.VMEM((BLK, DIM), jnp.float32),
        pltpu.VMEM((BLK, DIM), jnp.float32),
        pltpu.VMEM((BLK, DIM), jnp.float32),
        pltpu.VMEM((BLK, DIM), jnp.float32),
        pltpu.VMEM((BLK, DIM), jnp.float32),
        pltpu.VMEM((BLK, 16), jnp.float32),
        pltpu.VMEM_SHARED((N, DIM), jnp.float32),
        pltpu.VMEM_SHARED((N, 16), jnp.float32),
        pltpu.SemaphoreType.DMA,
        pltpu.SemaphoreType.DMA,
        pltpu.SemaphoreType.DMA,
        pltpu.SemaphoreType.DMA,
        pltpu.SemaphoreType.DMA,
        pltpu.SemaphoreType.DMA,
        pltpu.SemaphoreType.DMA,
        pltpu.SemaphoreType.DMA,
    ],
)


def kernel(x, edge_index, Wq, bq, Wk, bk, Wv, bv, Wo, bo):
    w_all = jnp.concatenate([Wq, Wk, Wv], axis=1)
    b_all = jnp.concatenate([bq, bk, bv]).reshape(1, 3 * DIM)
    q, k, v = _qkv_call(x, w_all, b_all)
    ei4 = edge_index.reshape(2, NW * NCH, CB, BLK)
    num, den = _edge_call(q, k, v, ei4)
    out = _out_call(num[0], num[1], den[0], den[1],
                    jnp.asarray(_K16), Wo, bo.reshape(1, DIM))
    return out
